# baseline (speedup 1.0000x reference)
import functools

import jax
import jax.numpy as jnp
from jax import lax
from jax.experimental import pallas as pl
from jax.experimental.pallas import tpu as pltpu
from jax.experimental.pallas import tpu_sc as plsc

B = 16384
NUM_CAT = 26
CAT_V = 100000
CAT_E = 16
ID_E = 32
NUM_NUM = 13
H = 128
OUT = 64
NPLANE = NUM_CAT * CAT_E

NC, NS = 2, 16
NW = NC * NS
BPW = B // NW
PPW = NPLANE // NW
IDCH = 4
HB = B // 2


_MESH = dict(core_axis_name="c", subcore_axis_name="s",
             num_cores=NC, num_subcores=NS)
_SC_PARAMS = dict(use_tc_tiling_on_sc=False, needs_layout_passes=False)


@functools.lru_cache(maxsize=None)
def _build_sc_cat():
    @functools.partial(
        pl.kernel,
        out_type=jax.ShapeDtypeStruct((NPLANE, B), jnp.float32),
        mesh=plsc.VectorSubcoreMesh(**_MESH),
        scratch_types=[
            pltpu.VMEM((B,), jnp.int32),
            pltpu.VMEM((CAT_V,), jnp.float32),
            pltpu.VMEM((HB,), jnp.float32),
            pltpu.SemaphoreType.DMA,
        ],
        compiler_params=pltpu.CompilerParams(**_SC_PARAMS),
    )
    def _sc_cat(catft_hbm, catpl_hbm, cat_out, idxf, plane, outv, sem):
        w = lax.axis_index("s") * NC + lax.axis_index("c")
        for k in range(PPW):
            p = w * PPW + k
            f = p // CAT_E
            if k == 0:
                pltpu.sync_copy(catft_hbm.at[f], idxf)
            else:
                fprev = (w * PPW + k - 1) // CAT_E

                @pl.when(f != fprev)
                def _reload():
                    pltpu.sync_copy(catft_hbm.at[f], idxf)
            pltpu.sync_copy(catpl_hbm.at[p], plane)
            for half in range(2):
                @pl.loop(0, HB // 16)
                def _gather(j):
                    iv = idxf[pl.ds(half * HB + j * 16, 16)]
                    outv[pl.ds(j * 16, 16)] = plsc.load_gather(plane, [iv])

                pltpu.sync_copy(outv, cat_out.at[p, pl.ds(half * HB, HB)])

    return _sc_cat


@functools.lru_cache(maxsize=None)
def _build_sc_id():
    @functools.partial(
        pl.kernel,
        out_type=jax.ShapeDtypeStruct((B, ID_E), jnp.bfloat16),
        mesh=plsc.VectorSubcoreMesh(**_MESH),
        scratch_types=[
            pltpu.VMEM((BPW,), jnp.int32),
            pltpu.VMEM((BPW, ID_E), jnp.bfloat16),
            pltpu.SemaphoreType.DMA,
        ],
        compiler_params=pltpu.CompilerParams(**_SC_PARAMS),
    )
    def _sc_id(ids_hbm, idt_hbm, id_out, idu, idrows, sem):
        w = lax.axis_index("s") * NC + lax.axis_index("c")
        ubase = w * BPW
        pltpu.sync_copy(ids_hbm.at[pl.ds(ubase, BPW)], idu)
        pltpu.async_copy(idt_hbm.at[idu], idrows, sem).wait()
        pltpu.sync_copy(idrows, id_out.at[pl.ds(ubase, BPW)])

    return _sc_id


BM = 2048


def _mlp_body(idt_ref, catt_ref, num_ref, w1a_ref, w1b_ref, w1c_ref, b1_ref,
              w2_ref, b2_ref, o_ref):
    dn = (((0,), (0,)), ((), ()))
    h = lax.dot_general(catt_ref[...], w1b_ref[...], dn,
                        preferred_element_type=jnp.float32)
    h += jnp.dot(idt_ref[...], w1a_ref[...],
                 preferred_element_type=jnp.float32)
    h += jnp.dot(num_ref[...], w1c_ref[...],
                 preferred_element_type=jnp.float32)
    h = jnp.maximum(h + b1_ref[...], 0.0)
    o = jnp.dot(h, w2_ref[...], preferred_element_type=jnp.float32)
    o += b2_ref[...]
    n = jnp.sqrt(jnp.sum(o * o, axis=1, keepdims=True))
    o_ref[...] = o / jnp.maximum(n, 1e-12)


def _mlp(id_t, cat_t, num_feats, w1a, w1b, w1c, b1, w2, b2):
    return pl.pallas_call(
        _mlp_body,
        grid=(B // BM,),
        in_specs=[
            pl.BlockSpec((BM, ID_E), lambda i: (i, 0)),
            pl.BlockSpec((NPLANE, BM), lambda i: (0, i)),
            pl.BlockSpec((BM, NUM_NUM), lambda i: (i, 0)),
            pl.BlockSpec((ID_E, H), lambda i: (0, 0)),
            pl.BlockSpec((NPLANE, H), lambda i: (0, 0)),
            pl.BlockSpec((NUM_NUM, H), lambda i: (0, 0)),
            pl.BlockSpec((1, H), lambda i: (0, 0)),
            pl.BlockSpec((H, OUT), lambda i: (0, 0)),
            pl.BlockSpec((1, OUT), lambda i: (0, 0)),
        ],
        out_specs=pl.BlockSpec((BM, OUT), lambda i: (i, 0)),
        out_shape=jax.ShapeDtypeStruct((B, OUT), jnp.float32),
        compiler_params=pltpu.CompilerParams(
            dimension_semantics=("arbitrary",)),
    )(id_t, cat_t, num_feats, w1a, w1b, w1c, b1, w2, b2)


def kernel(user_ids, user_cat_feats, user_numeric_feats, user_emb_table,
           cat_tables, W1, b1, W2, b2):
    idt = user_emb_table.astype(jnp.bfloat16)
    cat_pl = jnp.transpose(cat_tables, (0, 2, 1)).reshape(NPLANE, CAT_V)
    catft = jnp.transpose(user_cat_feats).astype(jnp.int32)
    cat_t = _build_sc_cat()(catft, cat_pl)
    id_t = _build_sc_id()(user_ids.astype(jnp.int32), idt)
    w1a = W1[:ID_E]
    w1b = W1[ID_E:ID_E + NPLANE]
    w1c = W1[ID_E + NPLANE:]
    return _mlp(id_t, cat_t, user_numeric_feats,
                w1a, w1b, w1c, b1.reshape(1, H), W2, b2.reshape(1, OUT))


# speedup vs baseline: 1.4139x; 1.4139x over previous
import functools

import jax
import jax.numpy as jnp
from jax import lax
from jax.experimental import pallas as pl
from jax.experimental.pallas import tpu as pltpu
from jax.experimental.pallas import tpu_sc as plsc

B = 16384
NUM_CAT = 26
CAT_V = 100000
CAT_E = 16
ID_E = 32
NUM_NUM = 13
H = 128
OUT = 64
NPLANE = NUM_CAT * CAT_E

NC, NS = 2, 16
NW = NC * NS
BPW = B // NW
PPW = NPLANE // NW
IDCH = 4
HB = B // 2


_MESH = dict(core_axis_name="c", subcore_axis_name="s",
             num_cores=NC, num_subcores=NS)
_SC_PARAMS = dict(use_tc_tiling_on_sc=False, needs_layout_passes=False)


@functools.lru_cache(maxsize=None)
def _build_sc_cat():
    @functools.partial(
        pl.kernel,
        out_type=jax.ShapeDtypeStruct((NPLANE, B), jnp.float32),
        mesh=plsc.VectorSubcoreMesh(**_MESH),
        scratch_types=[
            pltpu.VMEM((B,), jnp.int32),
            pltpu.VMEM((CAT_V,), jnp.float32),
            pltpu.VMEM((HB,), jnp.float32),
            pltpu.SemaphoreType.DMA,
        ],
        compiler_params=pltpu.CompilerParams(**_SC_PARAMS),
    )
    def _sc_cat(catft_hbm, catpl_hbm, cat_out, idxf, plane, outv, sem):
        w = lax.axis_index("s") * NC + lax.axis_index("c")
        for k in range(PPW):
            p = w * PPW + k
            f = p // CAT_E
            if k == 0:
                pltpu.sync_copy(catft_hbm.at[f], idxf)
            else:
                fprev = (w * PPW + k - 1) // CAT_E

                @pl.when(f != fprev)
                def _reload():
                    pltpu.sync_copy(catft_hbm.at[f], idxf)
            pltpu.sync_copy(catpl_hbm.at[p], plane)
            for half in range(2):
                @pl.loop(0, HB // 16)
                def _gather(j):
                    iv = idxf[pl.ds(half * HB + j * 16, 16)]
                    outv[pl.ds(j * 16, 16)] = plsc.load_gather(plane, [iv])

                pltpu.sync_copy(outv, cat_out.at[p, pl.ds(half * HB, HB)])

    return _sc_cat


@functools.lru_cache(maxsize=None)
def _build_sc_id():
    @functools.partial(
        pl.kernel,
        out_type=jax.ShapeDtypeStruct((B, ID_E), jnp.float32),
        mesh=plsc.VectorSubcoreMesh(**_MESH),
        scratch_types=[
            pltpu.VMEM((BPW,), jnp.int32),
            pltpu.VMEM((BPW, ID_E), jnp.float32),
            pltpu.SemaphoreType.DMA,
        ],
        compiler_params=pltpu.CompilerParams(**_SC_PARAMS),
    )
    def _sc_id(ids_hbm, idt_hbm, id_out, idu, idrows, sem):
        w = lax.axis_index("s") * NC + lax.axis_index("c")
        ubase = w * BPW
        pltpu.sync_copy(ids_hbm.at[pl.ds(ubase, BPW)], idu)
        pltpu.async_copy(idt_hbm.at[idu], idrows, sem).wait()
        pltpu.sync_copy(idrows, id_out.at[pl.ds(ubase, BPW)])

    return _sc_id


BM = 2048


def _mlp_body(idt_ref, catt_ref, num_ref, w1a_ref, w1b_ref, w1c_ref, b1_ref,
              w2_ref, b2_ref, o_ref):
    dn = (((0,), (0,)), ((), ()))
    h = lax.dot_general(catt_ref[...], w1b_ref[...], dn,
                        preferred_element_type=jnp.float32)
    h += jnp.dot(idt_ref[...], w1a_ref[...],
                 preferred_element_type=jnp.float32)
    h += jnp.dot(num_ref[...], w1c_ref[...],
                 preferred_element_type=jnp.float32)
    h = jnp.maximum(h + b1_ref[...], 0.0)
    o = jnp.dot(h, w2_ref[...], preferred_element_type=jnp.float32)
    o += b2_ref[...]
    n = jnp.sqrt(jnp.sum(o * o, axis=1, keepdims=True))
    o_ref[...] = o / jnp.maximum(n, 1e-12)


def _mlp(id_t, cat_t, num_feats, w1a, w1b, w1c, b1, w2, b2):
    return pl.pallas_call(
        _mlp_body,
        grid=(B // BM,),
        in_specs=[
            pl.BlockSpec((BM, ID_E), lambda i: (i, 0)),
            pl.BlockSpec((NPLANE, BM), lambda i: (0, i)),
            pl.BlockSpec((BM, NUM_NUM), lambda i: (i, 0)),
            pl.BlockSpec((ID_E, H), lambda i: (0, 0)),
            pl.BlockSpec((NPLANE, H), lambda i: (0, 0)),
            pl.BlockSpec((NUM_NUM, H), lambda i: (0, 0)),
            pl.BlockSpec((1, H), lambda i: (0, 0)),
            pl.BlockSpec((H, OUT), lambda i: (0, 0)),
            pl.BlockSpec((1, OUT), lambda i: (0, 0)),
        ],
        out_specs=pl.BlockSpec((BM, OUT), lambda i: (i, 0)),
        out_shape=jax.ShapeDtypeStruct((B, OUT), jnp.float32),
        compiler_params=pltpu.CompilerParams(
            dimension_semantics=("arbitrary",)),
    )(id_t, cat_t, num_feats, w1a, w1b, w1c, b1, w2, b2)


def kernel(user_ids, user_cat_feats, user_numeric_feats, user_emb_table,
           cat_tables, W1, b1, W2, b2):
    idt = user_emb_table
    cat_pl = jnp.transpose(cat_tables, (0, 2, 1)).reshape(NPLANE, CAT_V)
    catft = jnp.transpose(user_cat_feats).astype(jnp.int32)
    cat_t = _build_sc_cat()(catft, cat_pl)
    id_t = _build_sc_id()(user_ids.astype(jnp.int32), idt)
    w1a = W1[:ID_E]
    w1b = W1[ID_E:ID_E + NPLANE]
    w1c = W1[ID_E + NPLANE:]
    return _mlp(id_t, cat_t, user_numeric_feats,
                w1a, w1b, w1c, b1.reshape(1, H), W2, b2.reshape(1, OUT))
